# baseline (speedup 1.0000x reference)
import functools
import jax
import jax.numpy as jnp
from jax import lax
from jax.experimental import pallas as pl
from jax.experimental.pallas import tpu as pltpu
from jax.experimental.pallas import tpu_sc as plsc

K_DOCS = 1_000_000
D = 64
NW = 32
SHARE = 31_248
CHUNK = 256
NCHUNK = SHARE // CHUNK
TAIL = SHARE - NCHUNK * CHUNK
TAIL_OFF = NCHUNK * CHUNK
EXTRA_OFF = NW * SHARE
EXTRA = K_DOCS - EXTRA_OFF
NEG = -3.0e38
NCAND = 32
TOPK = 10
_IMAX = 2**31 - 1


def _insert4(s, g, cv, ci):
    out_v, out_i = [], []
    for j in range(4):
        gt = s > cv[j]
        out_v.append(jnp.where(gt, s, cv[j]))
        out_i.append(jnp.where(gt, g, ci[j]))
        s, g = jnp.where(gt, cv[j], s), jnp.where(gt, ci[j], g)
    return out_v, out_i


def _sc_body(q_hbm, d_hbm, vals_hbm, idx_hbm, buf, q1, vs, is_):
    wid = lax.axis_index("s") * 2 + lax.axis_index("c")
    base = wid * SHARE
    pltpu.sync_copy(q_hbm.at[0], q1)
    lane = lax.iota(jnp.int32, 16)
    q0 = q1[pl.ds(0, 16)]
    q1v = q1[pl.ds(16, 16)]
    q2 = q1[pl.ds(32, 16)]
    q3 = q1[pl.ds(48, 16)]

    def groups(goff, ngroups, carry):
        def group(g, c2):
            cv, ci = list(c2[0:4]), list(c2[4:8])
            dvec = jnp.zeros((16,), jnp.float32)
            nvec = jnp.ones((16,), jnp.float32)
            for dd in range(16):
                r = g * 16 + dd
                d0 = buf[r, pl.ds(0, 16)]
                d1 = buf[r, pl.ds(16, 16)]
                d2 = buf[r, pl.ds(32, 16)]
                d3 = buf[r, pl.ds(48, 16)]
                acc = d0 * q0 + d1 * q1v + d2 * q2 + d3 * q3
                nac = d0 * d0 + d1 * d1 + d2 * d2 + d3 * d3
                dvec = jnp.where(lane == dd, jnp.sum(acc), dvec)
                nvec = jnp.where(lane == dd, jnp.sum(nac), nvec)
            svec = dvec * jnp.abs(dvec) / nvec
            cv, ci = _insert4(svec, goff + g * 16 + lane, cv, ci)
            return tuple(cv) + tuple(ci)
        return lax.fori_loop(0, ngroups, group, carry)

    def chunk_loop(k, carry):
        pltpu.sync_copy(d_hbm.at[pl.ds(base + k * CHUNK, CHUNK)], buf)
        return groups(base + k * CHUNK, CHUNK // 16, carry)

    init = tuple(jnp.full((16,), NEG, jnp.float32) for _ in range(4)) + \
           tuple(jnp.zeros((16,), jnp.int32) for _ in range(4))
    carry = lax.fori_loop(0, NCHUNK, chunk_loop, init)

    pltpu.sync_copy(d_hbm.at[pl.ds(base + TAIL_OFF, TAIL)], buf.at[pl.ds(0, TAIL)])
    carry = groups(base + TAIL_OFF, TAIL // 16, carry)

    is_last = wid == NW - 1
    eoff = pl.multiple_of(jnp.where(is_last, EXTRA_OFF, 0), 8)
    pltpu.sync_copy(d_hbm.at[pl.ds(eoff, EXTRA)], buf.at[pl.ds(0, EXTRA)])
    ex = groups(eoff, EXTRA // 16, carry)
    carry = tuple(jnp.where(is_last, e, c) for e, c in zip(ex, carry))

    cv, ci = list(carry[0:4]), list(carry[4:8])
    for j in range(4):
        vs[pl.ds(j * 16, 16)] = cv[j]
        is_[pl.ds(j * 16, 16)] = ci[j]
    for j in range(4, 8):
        vs[pl.ds(j * 16, 16)] = jnp.full((16,), NEG, jnp.float32)
        is_[pl.ds(j * 16, 16)] = jnp.zeros((16,), jnp.int32)
    pltpu.sync_copy(vs, vals_hbm.at[wid])
    pltpu.sync_copy(is_, idx_hbm.at[wid])


def _sc_score(query, docs):
    mesh = plsc.VectorSubcoreMesh(core_axis_name="c", subcore_axis_name="s")
    fn = functools.partial(
        pl.kernel,
        out_type=[
            jax.ShapeDtypeStruct((NW, 128), jnp.float32),
            jax.ShapeDtypeStruct((NW, 128), jnp.int32),
        ],
        mesh=mesh,
        compiler_params=pltpu.CompilerParams(needs_layout_passes=False),
        scratch_types=[
            pltpu.VMEM((CHUNK, D), jnp.float32),
            pltpu.VMEM((D,), jnp.float32),
            pltpu.VMEM((128,), jnp.float32),
            pltpu.VMEM((128,), jnp.int32),
        ],
    )(_sc_body)
    return fn(query, docs)


def _merge_body(v_ref, g_ref, cand_ref):
    V = v_ref[...]
    G = g_ref[...]
    lane = lax.broadcasted_iota(jnp.int32, (1, NCAND), 1)
    cvec = jnp.zeros((1, NCAND), jnp.int32)
    for j in range(NCAND):
        m = jnp.max(V)
        gsel = jnp.min(jnp.where(V == m, G, _IMAX))
        cvec = jnp.where(lane == j, gsel, cvec)
        V = jnp.where((V == m) & (G == gsel), NEG, V)
    cand_ref[...] = cvec


def _rescore_body(cand_sref, q_ref, d_ref, vals_ref, idx_ref, vs_ref, gs_ref):
    i = pl.program_id(0)
    g = cand_sref[i]
    d8 = d_ref[...]
    q = q_ref[...]
    qn = jnp.sum(jnp.maximum(q * q, 1e-12))
    dot8 = jnp.sum(d8 * q, axis=1, keepdims=True)
    nrm8 = jnp.sum(jnp.maximum(d8 * d8, 1e-12), axis=1, keepdims=True)
    cos8 = dot8 / (jnp.sqrt(nrm8) * jnp.sqrt(qn))
    sub = g - (g // 8) * 8
    sel = lax.broadcasted_iota(jnp.int32, (8, 1), 0) == sub
    v = jnp.max(jnp.where(sel, cos8, -jnp.inf))

    @pl.when(i == 0)
    def _():
        vs_ref[...] = jnp.full((1, NCAND), -jnp.inf, jnp.float32)
        gs_ref[...] = jnp.zeros((1, NCAND), jnp.int32)

    laneC = lax.broadcasted_iota(jnp.int32, (1, NCAND), 1)
    vs_ref[...] = jnp.where(laneC == i, v, vs_ref[...])
    gs_ref[...] = jnp.where(laneC == i, g, gs_ref[...])

    @pl.when(i == NCAND - 1)
    def _():
        vs = vs_ref[...]
        gs = gs_ref[...]
        lane16 = lax.broadcasted_iota(jnp.int32, (1, 16), 1)
        vvec = jnp.full((1, 16), -jnp.inf, jnp.float32)
        ivec = jnp.zeros((1, 16), jnp.int32)
        for j in range(TOPK):
            m = jnp.max(vs)
            gsel = jnp.min(jnp.where(vs == m, gs, _IMAX))
            vvec = jnp.where(lane16 == j, m, vvec)
            ivec = jnp.where(lane16 == j, gsel, ivec)
            vs = jnp.where(gs == gsel, -jnp.inf, vs)
        vals_ref[...] = vvec
        idx_ref[...] = ivec


def kernel(query, docs):
    sc_vals, sc_idx = _sc_score(query, docs)

    cand = pl.pallas_call(
        _merge_body,
        out_shape=jax.ShapeDtypeStruct((1, NCAND), jnp.int32),
    )(sc_vals, sc_idx)

    vals, idx = pl.pallas_call(
        _rescore_body,
        grid_spec=pltpu.PrefetchScalarGridSpec(
            num_scalar_prefetch=1,
            grid=(NCAND,),
            in_specs=[
                pl.BlockSpec((1, D), lambda i, cand_s: (0, 0)),
                pl.BlockSpec((8, D), lambda i, cand_s: (cand_s[i] // 8, 0)),
            ],
            out_specs=[
                pl.BlockSpec((1, 16), lambda i, cand_s: (0, 0)),
                pl.BlockSpec((1, 16), lambda i, cand_s: (0, 0)),
            ],
            scratch_shapes=[
                pltpu.VMEM((1, NCAND), jnp.float32),
                pltpu.VMEM((1, NCAND), jnp.int32),
            ],
        ),
        out_shape=[
            jax.ShapeDtypeStruct((1, 16), jnp.float32),
            jax.ShapeDtypeStruct((1, 16), jnp.int32),
        ],
    )(cand.reshape(NCAND), query, docs)
    return vals[0, :TOPK], idx[0, :TOPK]


# speedup vs baseline: 1.2970x; 1.2970x over previous
import functools
import jax
import jax.numpy as jnp
from jax import lax
from jax.experimental import pallas as pl
from jax.experimental.pallas import tpu as pltpu
from jax.experimental.pallas import tpu_sc as plsc

K_DOCS = 1_000_000
D = 64
NW = 32
SHARE = 31_248
CHUNK = 256
NCHUNK = SHARE // CHUNK
TAIL = SHARE - NCHUNK * CHUNK
TAIL_OFF = NCHUNK * CHUNK
EXTRA_OFF = NW * SHARE
EXTRA = K_DOCS - EXTRA_OFF
NEG = -3.0e38
NCAND = 32
TOPK = 10
_IMAX = 2**31 - 1


def _insert4(s, g, cv, ci):
    out_v, out_i = [], []
    for j in range(4):
        gt = s > cv[j]
        out_v.append(jnp.where(gt, s, cv[j]))
        out_i.append(jnp.where(gt, g, ci[j]))
        s, g = jnp.where(gt, cv[j], s), jnp.where(gt, ci[j], g)
    return out_v, out_i


def _sc_body(q_hbm, d_hbm, vals_hbm, idx_hbm, buf, buf1, q1, vs, is_, sem0, sem1):
    wid = lax.axis_index("s") * 2 + lax.axis_index("c")
    base = wid * SHARE
    pltpu.sync_copy(q_hbm.at[0], q1)
    lane = lax.iota(jnp.int32, 16)
    q0 = q1[pl.ds(0, 16)]
    q1v = q1[pl.ds(16, 16)]
    q2 = q1[pl.ds(32, 16)]
    q3 = q1[pl.ds(48, 16)]

    def groups(goff, ngroups, carry):
        def group(g, c2):
            cv, ci = list(c2[0:4]), list(c2[4:8])
            dvec = jnp.zeros((16,), jnp.float32)
            nvec = jnp.ones((16,), jnp.float32)
            for dd in range(16):
                r = g * 16 + dd
                d0 = buf[r, pl.ds(0, 16)]
                d1 = buf[r, pl.ds(16, 16)]
                d2 = buf[r, pl.ds(32, 16)]
                d3 = buf[r, pl.ds(48, 16)]
                acc = d0 * q0 + d1 * q1v + d2 * q2 + d3 * q3
                nac = d0 * d0 + d1 * d1 + d2 * d2 + d3 * d3
                dvec = jnp.where(lane == dd, jnp.sum(acc), dvec)
                nvec = jnp.where(lane == dd, jnp.sum(nac), nvec)
            svec = dvec * jnp.abs(dvec) / nvec
            cv, ci = _insert4(svec, goff + g * 16 + lane, cv, ci)
            return tuple(cv) + tuple(ci)
        return lax.fori_loop(0, ngroups, group, carry)

    def groups_in(b, goff, ngroups, carry):
        def group(g, c2):
            cv, ci = list(c2[0:4]), list(c2[4:8])
            dvec = jnp.zeros((16,), jnp.float32)
            nvec = jnp.ones((16,), jnp.float32)
            for dd in range(16):
                r = g * 16 + dd
                d0 = b[r, pl.ds(0, 16)]
                d1 = b[r, pl.ds(16, 16)]
                d2 = b[r, pl.ds(32, 16)]
                d3 = b[r, pl.ds(48, 16)]
                acc = d0 * q0 + d1 * q1v + d2 * q2 + d3 * q3
                nac = d0 * d0 + d1 * d1 + d2 * d2 + d3 * d3
                dvec = jnp.where(lane == dd, jnp.sum(acc), dvec)
                nvec = jnp.where(lane == dd, jnp.sum(nac), nvec)
            svec = dvec * jnp.abs(dvec) / nvec
            cv, ci = _insert4(svec, goff + g * 16 + lane, cv, ci)
            return tuple(cv) + tuple(ci)
        return lax.fori_loop(0, ngroups, group, carry)

    def fire(c, b, sem):
        pltpu.async_copy(d_hbm.at[pl.ds(pl.multiple_of(c * CHUNK + base, 8),
                                        CHUNK)], b, sem)

    def drain(b, sem):
        pltpu.make_async_copy(d_hbm.at[pl.ds(base, CHUNK)], b, sem).wait()

    def chunk_pair(k2, carry):
        c = 2 * k2
        fire(c + 1, buf1, sem1)
        drain(buf, sem0)
        carry = groups_in(buf, base + c * CHUNK, CHUNK // 16, carry)
        fire(jnp.minimum(c + 2, NCHUNK - 2), buf, sem0)
        drain(buf1, sem1)
        carry = groups_in(buf1, base + (c + 1) * CHUNK, CHUNK // 16, carry)
        return carry

    init = tuple(jnp.full((16,), NEG, jnp.float32) for _ in range(4)) + \
           tuple(jnp.zeros((16,), jnp.int32) for _ in range(4))
    fire(0, buf, sem0)
    carry = lax.fori_loop(0, NCHUNK // 2, chunk_pair, init)
    drain(buf, sem0)

    pltpu.sync_copy(d_hbm.at[pl.ds(base + TAIL_OFF, TAIL)], buf.at[pl.ds(0, TAIL)])
    carry = groups(base + TAIL_OFF, TAIL // 16, carry)

    is_last = wid == NW - 1
    eoff = pl.multiple_of(jnp.where(is_last, EXTRA_OFF, 0), 8)
    pltpu.sync_copy(d_hbm.at[pl.ds(eoff, EXTRA)], buf.at[pl.ds(0, EXTRA)])
    ex = groups(eoff, EXTRA // 16, carry)
    carry = tuple(jnp.where(is_last, e, c) for e, c in zip(ex, carry))

    cv, ci = list(carry[0:4]), list(carry[4:8])
    for j in range(4):
        vs[pl.ds(j * 16, 16)] = cv[j]
        is_[pl.ds(j * 16, 16)] = ci[j]
    for j in range(4, 8):
        vs[pl.ds(j * 16, 16)] = jnp.full((16,), NEG, jnp.float32)
        is_[pl.ds(j * 16, 16)] = jnp.zeros((16,), jnp.int32)
    pltpu.sync_copy(vs, vals_hbm.at[wid])
    pltpu.sync_copy(is_, idx_hbm.at[wid])


def _sc_score(query, docs):
    mesh = plsc.VectorSubcoreMesh(core_axis_name="c", subcore_axis_name="s")
    fn = functools.partial(
        pl.kernel,
        out_type=[
            jax.ShapeDtypeStruct((NW, 128), jnp.float32),
            jax.ShapeDtypeStruct((NW, 128), jnp.int32),
        ],
        mesh=mesh,
        compiler_params=pltpu.CompilerParams(needs_layout_passes=False),
        scratch_types=[
            pltpu.VMEM((CHUNK, D), jnp.float32),
            pltpu.VMEM((CHUNK, D), jnp.float32),
            pltpu.VMEM((D,), jnp.float32),
            pltpu.VMEM((128,), jnp.float32),
            pltpu.VMEM((128,), jnp.int32),
            pltpu.SemaphoreType.DMA,
            pltpu.SemaphoreType.DMA,
        ],
    )(_sc_body)
    return fn(query, docs)


def _merge_body(v_ref, g_ref, cand_ref):
    V = v_ref[...]
    G = g_ref[...]
    lane = lax.broadcasted_iota(jnp.int32, (1, NCAND), 1)
    cvec = jnp.zeros((1, NCAND), jnp.int32)
    for j in range(NCAND):
        m = jnp.max(V)
        gsel = jnp.min(jnp.where(V == m, G, _IMAX))
        cvec = jnp.where(lane == j, gsel, cvec)
        V = jnp.where((V == m) & (G == gsel), NEG, V)
    cand_ref[...] = cvec


def _rescore_body(cand_sref, q_ref, d_ref, vals_ref, idx_ref, vs_ref, gs_ref):
    i = pl.program_id(0)
    g = cand_sref[i]
    d8 = d_ref[...]
    q = q_ref[...]
    qn = jnp.sum(jnp.maximum(q * q, 1e-12))
    dot8 = jnp.sum(d8 * q, axis=1, keepdims=True)
    nrm8 = jnp.sum(jnp.maximum(d8 * d8, 1e-12), axis=1, keepdims=True)
    cos8 = dot8 / (jnp.sqrt(nrm8) * jnp.sqrt(qn))
    sub = g - (g // 8) * 8
    sel = lax.broadcasted_iota(jnp.int32, (8, 1), 0) == sub
    v = jnp.max(jnp.where(sel, cos8, -jnp.inf))

    @pl.when(i == 0)
    def _():
        vs_ref[...] = jnp.full((1, NCAND), -jnp.inf, jnp.float32)
        gs_ref[...] = jnp.zeros((1, NCAND), jnp.int32)

    laneC = lax.broadcasted_iota(jnp.int32, (1, NCAND), 1)
    vs_ref[...] = jnp.where(laneC == i, v, vs_ref[...])
    gs_ref[...] = jnp.where(laneC == i, g, gs_ref[...])

    @pl.when(i == NCAND - 1)
    def _():
        vs = vs_ref[...]
        gs = gs_ref[...]
        lane16 = lax.broadcasted_iota(jnp.int32, (1, 16), 1)
        vvec = jnp.full((1, 16), -jnp.inf, jnp.float32)
        ivec = jnp.zeros((1, 16), jnp.int32)
        for j in range(TOPK):
            m = jnp.max(vs)
            gsel = jnp.min(jnp.where(vs == m, gs, _IMAX))
            vvec = jnp.where(lane16 == j, m, vvec)
            ivec = jnp.where(lane16 == j, gsel, ivec)
            vs = jnp.where(gs == gsel, -jnp.inf, vs)
        vals_ref[...] = vvec
        idx_ref[...] = ivec


def kernel(query, docs):
    sc_vals, sc_idx = _sc_score(query, docs)

    cand = pl.pallas_call(
        _merge_body,
        out_shape=jax.ShapeDtypeStruct((1, NCAND), jnp.int32),
    )(sc_vals, sc_idx)

    vals, idx = pl.pallas_call(
        _rescore_body,
        grid_spec=pltpu.PrefetchScalarGridSpec(
            num_scalar_prefetch=1,
            grid=(NCAND,),
            in_specs=[
                pl.BlockSpec((1, D), lambda i, cand_s: (0, 0)),
                pl.BlockSpec((8, D), lambda i, cand_s: (cand_s[i] // 8, 0)),
            ],
            out_specs=[
                pl.BlockSpec((1, 16), lambda i, cand_s: (0, 0)),
                pl.BlockSpec((1, 16), lambda i, cand_s: (0, 0)),
            ],
            scratch_shapes=[
                pltpu.VMEM((1, NCAND), jnp.float32),
                pltpu.VMEM((1, NCAND), jnp.int32),
            ],
        ),
        out_shape=[
            jax.ShapeDtypeStruct((1, 16), jnp.float32),
            jax.ShapeDtypeStruct((1, 16), jnp.int32),
        ],
    )(cand.reshape(NCAND), query, docs)
    return vals[0, :TOPK], idx[0, :TOPK]
